# baseline (speedup 1.0000x reference)
import jax
import jax.numpy as jnp
from jax.experimental import pallas as pl
from jax.experimental.pallas import tpu as pltpu


def _copy_body(src_ref, dst_ref, sem):
    copy = pltpu.make_async_copy(src_ref, dst_ref, sem)
    copy.start()
    copy.wait()


def kernel(med_embeddings):
    return pl.pallas_call(
        _copy_body,
        out_shape=jax.ShapeDtypeStruct(med_embeddings.shape, med_embeddings.dtype),
        in_specs=[pl.BlockSpec(memory_space=pltpu.MemorySpace.HBM)],
        out_specs=pl.BlockSpec(memory_space=pltpu.MemorySpace.HBM),
        scratch_shapes=[pltpu.SemaphoreType.DMA],
    )(med_embeddings)


# speedup vs baseline: 16.9912x; 16.9912x over previous
import jax
import jax.numpy as jnp
from jax.experimental import pallas as pl
from jax.experimental.pallas import tpu as pltpu

_BLOCK = 5_000


def _copy_body(src_ref, dst_ref):
    dst_ref[...] = src_ref[...]


def kernel(med_embeddings):
    n, d = med_embeddings.shape
    wide = med_embeddings.reshape(n * d // 128, 128)
    out = pl.pallas_call(
        _copy_body,
        grid=(wide.shape[0] // _BLOCK,),
        in_specs=[pl.BlockSpec((_BLOCK, 128), lambda i: (i, 0))],
        out_specs=pl.BlockSpec((_BLOCK, 128), lambda i: (i, 0)),
        out_shape=jax.ShapeDtypeStruct(wide.shape, wide.dtype),
    )(wide)
    return out.reshape(n, d)
